# Initial kernel scaffold; baseline (speedup 1.0000x reference)
import jax
import jax.numpy as jnp
from jax.experimental import pallas as pl


def kernel(x, edge_index, W1, b1, W2, b2):
    raise NotImplementedError("write your pallas kernel here")


# speedup vs baseline: 31.0957x; 31.0957x over previous
import functools

import jax
import jax.numpy as jnp
from jax import lax
from jax.experimental import pallas as pl
from jax.experimental.pallas import tpu as pltpu
from jax.experimental.pallas import tpu_sc as plsc

N = 10000
E = 320000
F_IN = 128
H = 16
C = 3

NC = 2
NS = 16
L = 16
NW = NC * NS

CHUNK = 128
CH_PER_W = 80
E_PAD = NW * CH_PER_W * CHUNK
N_ACC = 10112
RPS = N_ACC // NS

_mesh = plsc.VectorSubcoreMesh(core_axis_name="c", subcore_axis_name="s")
_sc_params = pltpu.CompilerParams(use_tc_tiling_on_sc=False)


@functools.partial(
    pl.kernel,
    out_type=jax.ShapeDtypeStruct((NC, N_ACC, L), jnp.float32),
    mesh=_mesh,
    compiler_params=_sc_params,
    scratch_types=[
        pltpu.VMEM((CH_PER_W, CHUNK), jnp.int32),
        pltpu.VMEM((CHUNK, L), jnp.float32),
        pltpu.VMEM((RPS, L), jnp.float32),
        pltpu.VMEM_SHARED((N_ACC, L), jnp.float32),
        pltpu.SemaphoreType.DMA,
    ],
)
def _sc_degree(dst_hbm, out_hbm, dst_v, ones_v, stage_v, acc_sh, sem):
    cid = lax.axis_index("c")
    sid = lax.axis_index("s")
    wid = cid * NS + sid

    zrow = jnp.zeros((L,), jnp.float32)
    orow = jnp.ones((L,), jnp.float32)

    @pl.loop(0, RPS)
    def _(i):
        stage_v[i, :] = zrow

    @pl.loop(0, CHUNK)
    def _(i):
        ones_v[i, :] = orow

    pltpu.sync_copy(stage_v, acc_sh.at[pl.ds(sid * RPS, RPS)])
    plsc.subcore_barrier()

    pltpu.sync_copy(dst_hbm.at[pl.ds(wid * CH_PER_W, CH_PER_W)], dst_v)

    @pl.loop(0, CH_PER_W)
    def _(j):
        pltpu.sync_copy(ones_v, acc_sh.at[dst_v.at[j]], add=True)

    plsc.subcore_barrier()
    pltpu.sync_copy(
        acc_sh.at[pl.ds(sid * RPS, RPS)],
        out_hbm.at[cid, pl.ds(sid * RPS, RPS)],
    )


@functools.partial(
    pl.kernel,
    out_type=jax.ShapeDtypeStruct((NC, N_ACC, H), jnp.float32),
    mesh=_mesh,
    compiler_params=_sc_params,
    scratch_types=[
        pltpu.VMEM((CH_PER_W, CHUNK), jnp.int32),
        pltpu.VMEM((CH_PER_W, CHUNK), jnp.int32),
        pltpu.VMEM((CHUNK, H), jnp.float32),
        pltpu.VMEM((CHUNK, H), jnp.float32),
        pltpu.VMEM((RPS, H), jnp.float32),
        pltpu.VMEM_SHARED((N_ACC, H), jnp.float32),
        pltpu.SemaphoreType.DMA,
        pltpu.SemaphoreType.DMA,
    ],
)
def _sc_aggregate(table_hbm, src_hbm, dst_hbm, out_hbm,
                  src_v, dst_v, rows_a, rows_b, stage_v, acc_sh, sem_a, sem_b):
    cid = lax.axis_index("c")
    sid = lax.axis_index("s")
    wid = cid * NS + sid

    zrow = jnp.zeros((L,), jnp.float32)

    @pl.loop(0, RPS)
    def _(i):
        stage_v[i, :] = zrow

    pltpu.sync_copy(stage_v, acc_sh.at[pl.ds(sid * RPS, RPS)])

    pltpu.sync_copy(src_hbm.at[pl.ds(wid * CH_PER_W, CH_PER_W)], src_v)
    pltpu.sync_copy(dst_hbm.at[pl.ds(wid * CH_PER_W, CH_PER_W)], dst_v)
    plsc.subcore_barrier()

    @pl.loop(0, CH_PER_W, step=2)
    def _(j):
        cpa = pltpu.async_copy(table_hbm.at[src_v.at[j]], rows_a, sem_a)
        cpb = pltpu.async_copy(table_hbm.at[src_v.at[j + 1]], rows_b, sem_b)
        cpa.wait()
        pltpu.sync_copy(rows_a, acc_sh.at[dst_v.at[j]], add=True)
        cpb.wait()
        pltpu.sync_copy(rows_b, acc_sh.at[dst_v.at[j + 1]], add=True)

    plsc.subcore_barrier()
    pltpu.sync_copy(
        acc_sh.at[pl.ds(sid * RPS, RPS)],
        out_hbm.at[cid, pl.ds(sid * RPS, RPS)],
    )


def _mm1_body(x_ref, w_ref, o_ref):
    o_ref[...] = jnp.dot(x_ref[...], w_ref[...],
                         preferred_element_type=jnp.float32)


def _prep_body(h1_ref, da_ref, db_ref, h1s_ref, dis_ref):
    deg = da_ref[...] + db_ref[...] + 1.0
    dis = lax.rsqrt(deg)
    dis_ref[...] = dis
    h1s_ref[...] = h1_ref[...] * dis


def _post1_body(aa_ref, ab_ref, h1s_ref, dis_ref, b1_ref, o_ref):
    pre = (aa_ref[...] + ab_ref[...] + h1s_ref[...]) * dis_ref[...] + b1_ref[...]
    o_ref[...] = jnp.maximum(pre, 0.0) * dis_ref[...]


def _final_body(aa_ref, ab_ref, o1s_ref, dis_ref, w2_ref, b2_ref, o_ref):
    h = (aa_ref[...] + ab_ref[...] + o1s_ref[...]) * dis_ref[...]
    o_ref[...] = jnp.dot(h, w2_ref[...],
                         preferred_element_type=jnp.float32) + b2_ref[...]


def kernel(x, edge_index, W1, b1, W2, b2):
    src = edge_index[0]
    dst = edge_index[1]
    pad = E_PAD - E
    src_p = jnp.concatenate(
        [src, jnp.zeros((pad,), jnp.int32)]).reshape(NW * CH_PER_W, CHUNK)
    dst_p = jnp.concatenate(
        [dst, jnp.full((pad,), N, jnp.int32)]).reshape(NW * CH_PER_W, CHUNK)

    deg_parts = _sc_degree(dst_p)

    h1 = pl.pallas_call(
        _mm1_body,
        out_shape=jax.ShapeDtypeStruct((N, H), jnp.float32),
    )(x, W1)

    h1s, dis = pl.pallas_call(
        _prep_body,
        out_shape=[
            jax.ShapeDtypeStruct((N, H), jnp.float32),
            jax.ShapeDtypeStruct((N, H), jnp.float32),
        ],
    )(h1, deg_parts[0, :N, :], deg_parts[1, :N, :])

    acc1 = _sc_aggregate(h1s, src_p, dst_p)

    o1s = pl.pallas_call(
        _post1_body,
        out_shape=jax.ShapeDtypeStruct((N, H), jnp.float32),
    )(acc1[0, :N, :], acc1[1, :N, :], h1s, dis, b1.reshape(1, H))

    acc2 = _sc_aggregate(o1s, src_p, dst_p)

    out = pl.pallas_call(
        _final_body,
        out_shape=jax.ShapeDtypeStruct((N, C), jnp.float32),
    )(acc2[0, :N, :], acc2[1, :N, :], o1s, dis, W2, b2.reshape(1, C))

    return out


# speedup vs baseline: 58.3298x; 1.8758x over previous
import functools

import jax
import jax.numpy as jnp
from jax import lax
from jax.experimental import pallas as pl
from jax.experimental.pallas import tpu as pltpu
from jax.experimental.pallas import tpu_sc as plsc

N = 10000
E = 320000
F_IN = 128
H = 16
C = 3

NC = 2
NS = 16
L = 16
NW = NC * NS

CHUNK = 128
EROWS = E // CHUNK
CH_BASE = EROWS // NW
CH_REM = EROWS % NW
NBUF = 4
N_ACC = 10112
RPS = N_ACC // NS

_mesh = plsc.VectorSubcoreMesh(core_axis_name="c", subcore_axis_name="s")
_sc_params = pltpu.CompilerParams(use_tc_tiling_on_sc=False)


@functools.partial(
    pl.kernel,
    out_type=jax.ShapeDtypeStruct((NC, N_ACC, L), jnp.float32),
    mesh=_mesh,
    compiler_params=_sc_params,
    scratch_types=[
        pltpu.VMEM((CH_BASE + 1, CHUNK), jnp.int32),
        pltpu.VMEM((CHUNK, L), jnp.float32),
        pltpu.VMEM((RPS, L), jnp.float32),
        pltpu.VMEM_SHARED((N_ACC, L), jnp.float32),
    ] + [pltpu.SemaphoreType.DMA] * NBUF,
)
def _sc_degree(e_hbm, out_hbm, dst_v, ones_v, stage_v, acc_sh, *ssem):
    cid = lax.axis_index("c")
    sid = lax.axis_index("s")
    wid = cid * NS + sid
    nch = jnp.where(wid < CH_REM, CH_BASE + 1, CH_BASE)

    zrow = jnp.zeros((L,), jnp.float32)
    orow = jnp.ones((L,), jnp.float32)

    @pl.loop(0, RPS)
    def _(i):
        stage_v[i, :] = zrow

    @pl.loop(0, CHUNK)
    def _(i):
        ones_v[i, :] = orow

    pltpu.sync_copy(stage_v, acc_sh.at[pl.ds(sid * RPS, RPS)])

    pltpu.sync_copy(e_hbm.at[1, pl.ds(wid * CH_BASE, CH_BASE)],
                    dst_v.at[pl.ds(0, CH_BASE)])

    @pl.when(wid < CH_REM)
    def _():
        pltpu.sync_copy(e_hbm.at[1, NW * CH_BASE + wid], dst_v.at[CH_BASE])

    plsc.subcore_barrier()

    def scat(j, b):
        return pltpu.make_async_copy(ones_v, acc_sh.at[dst_v.at[j]], ssem[b])

    for b in range(NBUF):
        scat(b, b).start(add=True)

    @pl.loop(0, CH_BASE + NBUF, step=NBUF)
    def _(jj):
        for b in range(NBUF):
            j4 = jj + NBUF + b

            @pl.when(j4 < nch)
            def _():
                scat(jj + b, b).wait()
                scat(j4, b).start(add=True)

    for b in range(NBUF):
        scat(0, b).wait()

    plsc.subcore_barrier()
    pltpu.sync_copy(
        acc_sh.at[pl.ds(sid * RPS, RPS)],
        out_hbm.at[cid, pl.ds(sid * RPS, RPS)],
    )


@functools.partial(
    pl.kernel,
    out_type=jax.ShapeDtypeStruct((NC, N_ACC, H), jnp.float32),
    mesh=_mesh,
    compiler_params=_sc_params,
    scratch_types=[
        pltpu.VMEM((CH_BASE + 1, CHUNK), jnp.int32),
        pltpu.VMEM((CH_BASE + 1, CHUNK), jnp.int32),
        pltpu.VMEM((RPS, H), jnp.float32),
        pltpu.VMEM_SHARED((N_ACC, H), jnp.float32),
    ] + [pltpu.VMEM((CHUNK, H), jnp.float32)] * NBUF
      + [pltpu.SemaphoreType.DMA] * (2 * NBUF),
)
def _sc_aggregate(table_hbm, e_hbm, out_hbm,
                  src_v, dst_v, stage_v, acc_sh, *bufs_sems):
    bufs = bufs_sems[:NBUF]
    gsem = bufs_sems[NBUF:2 * NBUF]
    ssem = bufs_sems[2 * NBUF:]

    cid = lax.axis_index("c")
    sid = lax.axis_index("s")
    wid = cid * NS + sid
    nch = jnp.where(wid < CH_REM, CH_BASE + 1, CH_BASE)

    zrow = jnp.zeros((L,), jnp.float32)

    @pl.loop(0, RPS)
    def _(i):
        stage_v[i, :] = zrow

    pltpu.sync_copy(stage_v, acc_sh.at[pl.ds(sid * RPS, RPS)])

    pltpu.sync_copy(e_hbm.at[0, pl.ds(wid * CH_BASE, CH_BASE)],
                    src_v.at[pl.ds(0, CH_BASE)])
    pltpu.sync_copy(e_hbm.at[1, pl.ds(wid * CH_BASE, CH_BASE)],
                    dst_v.at[pl.ds(0, CH_BASE)])

    @pl.when(wid < CH_REM)
    def _():
        pltpu.sync_copy(e_hbm.at[0, NW * CH_BASE + wid], src_v.at[CH_BASE])
        pltpu.sync_copy(e_hbm.at[1, NW * CH_BASE + wid], dst_v.at[CH_BASE])

    plsc.subcore_barrier()

    def gat(j, b):
        return pltpu.make_async_copy(
            table_hbm.at[src_v.at[j]], bufs[b], gsem[b])

    def scat(j, b):
        return pltpu.make_async_copy(bufs[b], acc_sh.at[dst_v.at[j]], ssem[b])

    for b in range(NBUF):
        gat(b, b).start()

    @pl.loop(0, CH_BASE + NBUF, step=NBUF)
    def _(jj):
        for b in range(NBUF):
            j = jj + b

            @pl.when(j < nch)
            def _():
                gat(j, b).wait()
                scat(j, b).start(add=True)
        for b in range(NBUF):
            j4 = jj + NBUF + b

            @pl.when(j4 < nch)
            def _():
                scat(jj + b, b).wait()
                gat(j4, b).start()

    for b in range(NBUF):
        scat(0, b).wait()

    plsc.subcore_barrier()
    pltpu.sync_copy(
        acc_sh.at[pl.ds(sid * RPS, RPS)],
        out_hbm.at[cid, pl.ds(sid * RPS, RPS)],
    )


def _mm1_body(x_ref, w_ref, o_ref):
    o_ref[...] = jnp.dot(x_ref[...], w_ref[...],
                         preferred_element_type=jnp.float32)


def _prep_body(h1_ref, dp_ref, h1s_ref, dis_ref):
    deg = dp_ref[0, :N, :] + dp_ref[1, :N, :] + 1.0
    dis = lax.rsqrt(deg)
    dis_ref[...] = dis
    h1s_ref[...] = h1_ref[...] * dis


def _post1_body(acc_ref, h1s_ref, dis_ref, b1_ref, o_ref):
    agg = acc_ref[0, :N, :] + acc_ref[1, :N, :] + h1s_ref[...]
    pre = agg * dis_ref[...] + b1_ref[...]
    o_ref[...] = jnp.maximum(pre, 0.0) * dis_ref[...]


def _final_body(acc_ref, o1s_ref, dis_ref, w2_ref, b2_ref, o_ref):
    agg = acc_ref[0, :N, :] + acc_ref[1, :N, :] + o1s_ref[...]
    h = agg * dis_ref[...]
    o_ref[...] = jnp.dot(h, w2_ref[...],
                         preferred_element_type=jnp.float32) + b2_ref[...]


def kernel(x, edge_index, W1, b1, W2, b2):
    e3 = edge_index.reshape(2, EROWS, CHUNK)

    deg_parts = _sc_degree(e3)

    h1 = pl.pallas_call(
        _mm1_body,
        out_shape=jax.ShapeDtypeStruct((N, H), jnp.float32),
    )(x, W1)

    h1s, dis = pl.pallas_call(
        _prep_body,
        out_shape=[
            jax.ShapeDtypeStruct((N, H), jnp.float32),
            jax.ShapeDtypeStruct((N, H), jnp.float32),
        ],
    )(h1, deg_parts)

    acc1 = _sc_aggregate(h1s, e3)

    o1s = pl.pallas_call(
        _post1_body,
        out_shape=jax.ShapeDtypeStruct((N, H), jnp.float32),
    )(acc1, h1s, dis, b1.reshape(1, H))

    acc2 = _sc_aggregate(o1s, e3)

    out = pl.pallas_call(
        _final_body,
        out_shape=jax.ShapeDtypeStruct((N, C), jnp.float32),
    )(acc2, o1s, dis, W2, b2.reshape(1, C))

    return out
